# baseline (speedup 1.0000x reference)
from sc_try import sc_kernel


def kernel(x, pos_embed):
    return sc_kernel(x, pos_embed)


# speedup vs baseline: 1.0872x; 1.0872x over previous
from sc_try2 import sc_kernel2


def kernel(x, pos_embed):
    return sc_kernel2(x, pos_embed)


# speedup vs baseline: 7.2060x; 6.6283x over previous
import jax
import jax.numpy as jnp
from jax.experimental import pallas as pl


def _add_kernel(x_ref, pe_ref, o_ref):
    o_ref[...] = x_ref[...] + pe_ref[...]


def kernel(x, pos_embed):
    B, S, D = x.shape
    pe = pos_embed[:S]
    BS = 2048
    grid = (S // BS, B)
    return pl.pallas_call(
        _add_kernel,
        grid=grid,
        in_specs=[
            pl.BlockSpec((1, BS, D), lambda s, b: (b, s, 0)),
            pl.BlockSpec((BS, D), lambda s, b: (s, 0)),
        ],
        out_specs=pl.BlockSpec((1, BS, D), lambda s, b: (b, s, 0)),
        out_shape=jax.ShapeDtypeStruct((B, S, D), x.dtype),
    )(x, pe)
